# baseline (speedup 1.0000x reference)
import functools

import jax
import jax.numpy as jnp
from jax import lax
from jax.experimental import pallas as pl
from jax.experimental.pallas import tpu as pltpu
from jax.experimental.pallas import tpu_sc as plsc

N = 10000
E = 320000
D = 128
L = 4
D_OUT = 51
EPS = 1e-5

NC = 2
NS = 16
NW = NC * NS
EW = E // NW
B = 40
SUP = 50
NSUPER = EW // (B * SUP)
NPAD = 10112
RPS = NPAD // NS


def _bn(h, g, b):
    mu = jnp.mean(h, axis=0, keepdims=True)
    hc = h - mu
    var = jnp.mean(hc * hc, axis=0, keepdims=True)
    return g * hc * lax.rsqrt(var + EPS) + b


def _post(a, g, b):
    o = jax.nn.relu(_bn(a, g, b))
    nrm = jnp.sqrt(jnp.sum(o * o, axis=-1, keepdims=True))
    return o / (nrm + 1e-12)


def _proj(h, w, b):
    return jnp.dot(h, w, preferred_element_type=jnp.float32) + b


def _pre_body(x, wpre, bpre, gpre, bepre, wkn, wq, wv, ws, bkn, bq, bv, bs,
              h_o, kn_o, qv_o, s_o):
    h = _proj(x[...], wpre[...], bpre[...])
    h = _post(h, gpre[...], bepre[...])
    h_o[...] = h
    kn_o[...] = _proj(h, wkn[...], bkn[...])
    qv_o[:, 0:D] = _proj(h, wq[...], bq[...])
    qv_o[:, D:2 * D] = _proj(h, wv[...], bv[...])
    s_o[...] = _proj(h, ws[...], bs[...])


_kqv_shapes = (
    jax.ShapeDtypeStruct((N, D), jnp.float32),
    jax.ShapeDtypeStruct((N, D), jnp.float32),
    jax.ShapeDtypeStruct((N, 2 * D), jnp.float32),
    jax.ShapeDtypeStruct((N, D), jnp.float32),
)

_pre_call = pl.pallas_call(_pre_body, out_shape=_kqv_shapes)


def _mid_body(aggr, h, s, g, be, wkn, wq, wv, ws, bkn, bq, bv, bs,
              h_o, kn_o, qv_o, s_o):
    a = aggr[0:N, :] + aggr[NPAD:NPAD + N, :] + s[...]
    h2 = h[...] + _post(a, g[...], be[...])
    h_o[...] = h2
    kn_o[...] = _proj(h2, wkn[...], bkn[...])
    qv_o[:, 0:D] = _proj(h2, wq[...], bq[...])
    qv_o[:, D:2 * D] = _proj(h2, wv[...], bv[...])
    s_o[...] = _proj(h2, ws[...], bs[...])


_mid_call = pl.pallas_call(_mid_body, out_shape=_kqv_shapes)


def _final_body(aggr, h, s, g, be, wh1, bh1, wh2, bh2, pred_o):
    a = aggr[0:N, :] + aggr[NPAD:NPAD + N, :] + s[...]
    h2 = h[...] + _post(a, g[...], be[...])
    z = jax.nn.relu(_proj(h2, wh1[...], bh1[...]))
    pred_o[...] = _proj(z, wh2[...], bh2[...])


_final_call = pl.pallas_call(
    _final_body,
    out_shape=jax.ShapeDtypeStruct((N, D_OUT), jnp.float32),
)


_sc_mesh = plsc.VectorSubcoreMesh(core_axis_name="c", subcore_axis_name="s")


@functools.partial(
    pl.kernel,
    out_type=jax.ShapeDtypeStruct((2 * NPAD, D), jnp.float32),
    mesh=_sc_mesh,
    scratch_types=[
        pltpu.VMEM((SUP, B), jnp.int32),
        pltpu.VMEM((SUP, B), jnp.int32),
        pltpu.VMEM((B, D), jnp.float32),
        pltpu.VMEM((B, 2 * D), jnp.float32),
        pltpu.VMEM((B, D), jnp.float32),
        pltpu.VMEM((B, 2 * D), jnp.float32),
        pltpu.VMEM((B, D), jnp.float32),
        pltpu.VMEM_SHARED((NPAD, D), jnp.float32),
        pltpu.SemaphoreType.DMA,
        pltpu.SemaphoreType.DMA,
    ],
)
def _edge_call(kn_hbm, qv_hbm, src_hbm, dst_hbm, out_hbm,
               srcv, dstv, kn0, qv0, kn1, qv1, msg, aggr,
               gsem0, gsem1):
    cid = lax.axis_index("c")
    sid = lax.axis_index("s")
    wid = sid * NC + cid
    base = wid * NSUPER

    gslots = ((kn0, qv0, gsem0), (kn1, qv1, gsem1))

    zeros16 = jnp.zeros((16,), jnp.float32)

    def zrow(r, _):
        for t in range(D // 16):
            msg[r, pl.ds(t * 16, 16)] = zeros16
        return 0

    lax.fori_loop(0, B, zrow, 0)

    def zcopy(t, _):
        pltpu.sync_copy(msg.at[pl.ds(0, 8)],
                        aggr.at[pl.ds(sid * RPS + t * 8, 8)])
        return 0

    lax.fori_loop(0, RPS // 8, zcopy, 0)
    plsc.subcore_barrier()

    def issue(j, slot):
        kn, qv, sem = gslots[slot]
        pltpu.async_copy(kn_hbm.at[dstv.at[j]], kn, sem)
        pltpu.async_copy(qv_hbm.at[srcv.at[j]], qv, sem)

    def wait_gathers(j, slot):
        kn, qv, sem = gslots[slot]
        pltpu.make_async_copy(kn_hbm.at[dstv.at[j]], kn, sem).wait()
        pltpu.make_async_copy(qv_hbm.at[srcv.at[j]], qv, sem).wait()

    def compute(slot):
        kn, qv, _ = gslots[slot]

        def row(r, _):
            for t in range(D // 16):
                sl = pl.ds(t * 16, 16)
                ex = jnp.exp(kn[r, sl] - qv[r, sl])
                msg[r, sl] = qv[r, pl.ds(D + t * 16, 16)] / (1.0 + ex)
            return 0

        lax.fori_loop(0, B, row, 0)

    def scatter(j):
        pltpu.sync_copy(msg, aggr.at[dstv.at[j]], add=True)

    for g in range(NSUPER):
        pltpu.sync_copy(src_hbm.at[base + g], srcv)
        pltpu.sync_copy(dst_hbm.at[base + g], dstv)

        issue(0, 0)
        issue(1, 1)

        def body(t, _):
            for slot in range(2):
                jj = 2 * t + slot
                wait_gathers(jj, slot)
                compute(slot)
                issue(jj + 2, slot)
                scatter(jj)
            return 0

        lax.fori_loop(0, SUP // 2 - 1, body, 0)
        for slot in range(2):
            jj = SUP - 2 + slot
            wait_gathers(jj, slot)
            compute(slot)
            scatter(jj)

    plsc.subcore_barrier()

    r0 = sid * RPS
    pltpu.sync_copy(aggr.at[pl.ds(r0, RPS)],
                    out_hbm.at[pl.ds(cid * NPAD + r0, RPS)])


def kernel(x, edge_index, W_pre, b_pre, g_pre, be_pre, Wk, Wq, Wv, Ws,
           bk, bq, bv, bs, gamma, beta, Wh1, bh1, Wh2, bh2):
    src = edge_index[0].reshape(NW * NSUPER, SUP, B)
    dst = edge_index[1].reshape(NW * NSUPER, SUP, B)
    r = lambda t: t.reshape(1, -1)

    h, kn, qv, s = _pre_call(
        x, W_pre, r(b_pre), r(g_pre), r(be_pre),
        -Wk[0], Wq[0], Wv[0], Ws[0], r(-bk[0]), r(bq[0]), r(bv[0]), r(bs[0]))

    for l in range(L):
        aggr2 = _edge_call(kn, qv, src, dst)
        if l < L - 1:
            h, kn, qv, s = _mid_call(
                aggr2, h, s, r(gamma[l]), r(beta[l]),
                -Wk[l + 1], Wq[l + 1], Wv[l + 1], Ws[l + 1],
                r(-bk[l + 1]), r(bq[l + 1]), r(bv[l + 1]), r(bs[l + 1]))
        else:
            pred = _final_call(
                aggr2, h, s, r(gamma[l]), r(beta[l]),
                Wh1, r(bh1), Wh2, r(bh2))
    return pred


# speedup vs baseline: 1.0005x; 1.0005x over previous
import functools

import jax
import jax.numpy as jnp
from jax import lax
from jax.experimental import pallas as pl
from jax.experimental.pallas import tpu as pltpu
from jax.experimental.pallas import tpu_sc as plsc

N = 10000
E = 320000
D = 128
L = 4
D_OUT = 51
EPS = 1e-5

NC = 2
NS = 16
NW = NC * NS
EW = E // NW
B = 40
SUP = 50
NSUPER = EW // (B * SUP)
NPAD = 10112
RPS = NPAD // NS


def _bn(h, g, b):
    mu = jnp.mean(h, axis=0, keepdims=True)
    hc = h - mu
    var = jnp.mean(hc * hc, axis=0, keepdims=True)
    return g * hc * lax.rsqrt(var + EPS) + b


def _post(a, g, b):
    o = jax.nn.relu(_bn(a, g, b))
    nrm = jnp.sqrt(jnp.sum(o * o, axis=-1, keepdims=True))
    return o / (nrm + 1e-12)


def _proj(h, w, b):
    return jnp.dot(h, w, preferred_element_type=jnp.float32) + b


def _pre_body(x, wpre, bpre, gpre, bepre, wkn, wq, wv, ws, bkn, bq, bv, bs,
              h_o, kn_o, qv_o, s_o):
    h = _proj(x[...], wpre[...], bpre[...])
    h = _post(h, gpre[...], bepre[...])
    h_o[...] = h
    kn_o[...] = _proj(h, wkn[...], bkn[...])
    qv_o[:, 0:D] = _proj(h, wq[...], bq[...])
    qv_o[:, D:2 * D] = _proj(h, wv[...], bv[...])
    s_o[...] = _proj(h, ws[...], bs[...])


_kqv_shapes = (
    jax.ShapeDtypeStruct((N, D), jnp.float32),
    jax.ShapeDtypeStruct((N, D), jnp.float32),
    jax.ShapeDtypeStruct((N, 2 * D), jnp.float32),
    jax.ShapeDtypeStruct((N, D), jnp.float32),
)

_pre_call = pl.pallas_call(_pre_body, out_shape=_kqv_shapes)


def _mid_body(aggr, h, s, g, be, wkn, wq, wv, ws, bkn, bq, bv, bs,
              h_o, kn_o, qv_o, s_o):
    a = aggr[0:N, :] + aggr[NPAD:NPAD + N, :] + s[...]
    h2 = h[...] + _post(a, g[...], be[...])
    h_o[...] = h2
    kn_o[...] = _proj(h2, wkn[...], bkn[...])
    qv_o[:, 0:D] = _proj(h2, wq[...], bq[...])
    qv_o[:, D:2 * D] = _proj(h2, wv[...], bv[...])
    s_o[...] = _proj(h2, ws[...], bs[...])


_mid_call = pl.pallas_call(_mid_body, out_shape=_kqv_shapes)


def _final_body(aggr, h, s, g, be, wh1, bh1, wh2, bh2, pred_o):
    a = aggr[0:N, :] + aggr[NPAD:NPAD + N, :] + s[...]
    h2 = h[...] + _post(a, g[...], be[...])
    z = jax.nn.relu(_proj(h2, wh1[...], bh1[...]))
    pred_o[...] = _proj(z, wh2[...], bh2[...])


_final_call = pl.pallas_call(
    _final_body,
    out_shape=jax.ShapeDtypeStruct((N, D_OUT), jnp.float32),
)


_sc_mesh = plsc.VectorSubcoreMesh(core_axis_name="c", subcore_axis_name="s")


@functools.partial(
    pl.kernel,
    out_type=jax.ShapeDtypeStruct((2 * NPAD, D), jnp.float32),
    mesh=_sc_mesh,
    scratch_types=[
        pltpu.VMEM((SUP, B), jnp.int32),
        pltpu.VMEM((SUP, B), jnp.int32),
        pltpu.VMEM((B, D), jnp.float32),
        pltpu.VMEM((B, 2 * D), jnp.float32),
        pltpu.VMEM((B, D), jnp.float32),
        pltpu.VMEM((B, 2 * D), jnp.float32),
        pltpu.VMEM((B, D), jnp.float32),
        pltpu.VMEM_SHARED((NPAD, D), jnp.float32),
        pltpu.SemaphoreType.DMA,
        pltpu.SemaphoreType.DMA,
    ],
)
def _edge_call(kn_hbm, qv_hbm, src_hbm, dst_hbm, out_hbm,
               srcv, dstv, kn0, qv0, kn1, qv1, msg, aggr,
               gsem0, gsem1):
    cid = lax.axis_index("c")
    sid = lax.axis_index("s")
    wid = sid * NC + cid
    base = wid * NSUPER

    gslots = ((kn0, qv0, gsem0), (kn1, qv1, gsem1))

    zeros16 = jnp.zeros((16,), jnp.float32)

    def zrow(r, _):
        for t in range(D // 16):
            msg[r, pl.ds(t * 16, 16)] = zeros16
        return 0

    lax.fori_loop(0, B, zrow, 0)

    def zcopy(t, _):
        pltpu.sync_copy(msg.at[pl.ds(0, 8)],
                        aggr.at[pl.ds(sid * RPS + t * 8, 8)])
        return 0

    lax.fori_loop(0, RPS // 8, zcopy, 0)
    plsc.subcore_barrier()

    def issue(j, slot):
        kn, qv, sem = gslots[slot]
        pltpu.async_copy(kn_hbm.at[dstv.at[j]], kn, sem)
        pltpu.async_copy(qv_hbm.at[srcv.at[j]], qv, sem)

    def wait_gathers(j, slot):
        kn, qv, sem = gslots[slot]
        pltpu.make_async_copy(kn_hbm.at[dstv.at[j]], kn, sem).wait()
        pltpu.make_async_copy(qv_hbm.at[srcv.at[j]], qv, sem).wait()

    def compute(slot):
        kn, qv, _ = gslots[slot]

        def row(r, _):
            for t in range(D // 16):
                sl = pl.ds(t * 16, 16)
                ex = jnp.exp(kn[r, sl] - qv[r, sl])
                msg[r, sl] = qv[r, pl.ds(D + t * 16, 16)] / (1.0 + ex)
            return 0

        lax.fori_loop(0, B, row, 0)

    def scatter(j):
        pltpu.sync_copy(msg, aggr.at[dstv.at[j]], add=True)

    for g in range(NSUPER):
        pltpu.sync_copy(src_hbm.at[base + g], srcv)
        pltpu.sync_copy(dst_hbm.at[base + g], dstv)

        issue(0, 0)
        issue(1, 1)

        def body(t, _):
            for slot in range(2):
                jj = 2 * t + slot
                wait_gathers(jj, slot)
                compute(slot)
                scatter(jj)
                issue(jj + 2, slot)
            return 0

        lax.fori_loop(0, SUP // 2 - 1, body, 0)
        for slot in range(2):
            jj = SUP - 2 + slot
            wait_gathers(jj, slot)
            compute(slot)
            scatter(jj)

    plsc.subcore_barrier()

    r0 = sid * RPS
    pltpu.sync_copy(aggr.at[pl.ds(r0, RPS)],
                    out_hbm.at[pl.ds(cid * NPAD + r0, RPS)])


def kernel(x, edge_index, W_pre, b_pre, g_pre, be_pre, Wk, Wq, Wv, Ws,
           bk, bq, bv, bs, gamma, beta, Wh1, bh1, Wh2, bh2):
    src = edge_index[0].reshape(NW * NSUPER, SUP, B)
    dst = edge_index[1].reshape(NW * NSUPER, SUP, B)
    r = lambda t: t.reshape(1, -1)

    h, kn, qv, s = _pre_call(
        x, W_pre, r(b_pre), r(g_pre), r(be_pre),
        -Wk[0], Wq[0], Wv[0], Ws[0], r(-bk[0]), r(bq[0]), r(bv[0]), r(bs[0]))

    for l in range(L):
        aggr2 = _edge_call(kn, qv, src, dst)
        if l < L - 1:
            h, kn, qv, s = _mid_call(
                aggr2, h, s, r(gamma[l]), r(beta[l]),
                -Wk[l + 1], Wq[l + 1], Wv[l + 1], Ws[l + 1],
                r(-bk[l + 1]), r(bq[l + 1]), r(bv[l + 1]), r(bs[l + 1]))
        else:
            pred = _final_call(
                aggr2, h, s, r(gamma[l]), r(beta[l]),
                Wh1, r(bh1), Wh2, r(bh2))
    return pred
